# Initial kernel scaffold; baseline (speedup 1.0000x reference)
import jax
import jax.numpy as jnp
from jax.experimental import pallas as pl


def kernel(features, codebook_weight):
    raise NotImplementedError("write your pallas kernel here")


# speedup vs baseline: 1.1037x; 1.1037x over previous
import functools

import jax
import jax.numpy as jnp
from jax.experimental import pallas as pl
from jax.experimental.pallas import tpu as pltpu
from jax.experimental.pallas import tpu_sc as plsc

_VOCAB = 8192
_WIDTH = 32
_BETA = 0.25
_TOK_BLK = 256
_GATHER_WIN = 128
_GATHER_PAD = 128


def _prep_body(cb_ref, cbn_ref, cbt_ref):
    cb = cb_ref[...]
    ss = jnp.sum(cb * cb, axis=1, keepdims=True)
    r = 1.0 / jnp.maximum(jnp.sqrt(ss), 1e-12)
    cbn = cb * r
    pad = jnp.zeros((_VOCAB, _GATHER_PAD - _WIDTH), jnp.float32)
    cbn_ref[...] = jnp.concatenate([cbn, pad], axis=1)
    cbt_ref[...] = cbn.T


def _argmax_body(nblk, f_ref, cbt_ref, idx_ref, used_ref, stats_ref):
    i = pl.program_id(0)
    f = f_ref[...]
    ss = jnp.sum(f * f, axis=1, keepdims=True)
    r = 1.0 / jnp.maximum(jnp.sqrt(ss), 1e-12)
    fn = f * r
    sims = jnp.dot(fn, cbt_ref[...],
                   preferred_element_type=jnp.float32)
    m = jnp.max(sims, axis=1, keepdims=True)
    iota = jax.lax.broadcasted_iota(jnp.int32, sims.shape, 1)
    idx = jnp.min(jnp.where(sims >= m, iota, _VOCAB), axis=1)
    idx_ref[0, 0, :] = idx

    mask = jnp.max((idx[:, None] == iota).astype(jnp.float32), axis=0)
    mask8 = jnp.broadcast_to(mask[None, :], (8, _VOCAB))
    prev = jnp.where(i == 0, jnp.zeros_like(mask8), used_ref[...])
    used_ref[...] = jnp.maximum(prev, mask8)

    @pl.when(i == 0)
    def _():
        stats_ref[0, 0] = 0.0
        stats_ref[0, 1] = 0.0

    stats_ref[0, 0] += jnp.sum(m)
    stats_ref[0, 1] += jnp.sum(fn * fn)

    @pl.when(i == nblk - 1)
    def _():
        n = jnp.float32(nblk * _TOK_BLK)
        s_sum = stats_ref[0, 0]
        nf2_sum = stats_ref[0, 1]
        stats_ref[0, 2] = ((1.0 + _BETA) * (nf2_sum - 2.0 * s_sum + n)
                           / (n * _WIDTH))
        stats_ref[0, 3] = 100.0 * jnp.sum(used_ref[0, :]) / _VOCAB


def _tc_argmax(f, cbn_t):
    n = f.shape[0]
    nblk = n // _TOK_BLK
    body = functools.partial(_argmax_body, nblk)
    return pl.pallas_call(
        body,
        grid=(nblk,),
        in_specs=[
            pl.BlockSpec((_TOK_BLK, _WIDTH), lambda i: (i, 0)),
            pl.BlockSpec((_WIDTH, _VOCAB), lambda i: (0, 0)),
        ],
        out_specs=[
            pl.BlockSpec((1, 1, _TOK_BLK), lambda i: (i, 0, 0)),
            pl.BlockSpec((8, _VOCAB), lambda i: (0, 0)),
            pl.BlockSpec(memory_space=pltpu.SMEM),
        ],
        out_shape=[
            jax.ShapeDtypeStruct((nblk, 1, _TOK_BLK), jnp.int32),
            jax.ShapeDtypeStruct((8, _VOCAB), jnp.float32),
            jax.ShapeDtypeStruct((1, 4), jnp.float32),
        ],
    )(f, cbn_t)


def _prep(cb):
    return pl.pallas_call(
        _prep_body,
        out_shape=[
            jax.ShapeDtypeStruct((_VOCAB, _GATHER_PAD), jnp.float32),
            jax.ShapeDtypeStruct((_WIDTH, _VOCAB), jnp.float32),
        ],
    )(cb)


def _sc_gather(cbn, idx2d):
    n = idx2d.shape[1]
    mesh = plsc.VectorSubcoreMesh(core_axis_name="c", subcore_axis_name="s")

    @functools.partial(
        pl.kernel,
        out_type=jax.ShapeDtypeStruct((n, _GATHER_PAD), jnp.float32),
        mesh=mesh,
    )
    def gather_kernel(cb_hbm, i_hbm, o_hbm):
        def body(i_vmem, o_vmem):
            pltpu.sync_copy(cb_hbm.at[i_vmem.at[0]], o_vmem)

        pltpu.emit_pipeline(
            body,
            grid=(n // _GATHER_WIN,),
            in_specs=[pl.BlockSpec((1, _GATHER_WIN), lambda i: (0, i))],
            out_specs=[pl.BlockSpec((_GATHER_WIN, _GATHER_PAD),
                                    lambda i: (i, 0))],
            core_axis_name="s",
            dimension_semantics=(pltpu.PARALLEL,),
        )(i_hbm, o_hbm)

    return gather_kernel(cbn, idx2d)


def kernel(features, codebook_weight):
    b, l, c = features.shape
    f = features.reshape(-1, c)
    cbn, cbn_t = _prep(codebook_weight)
    idx3, _used, stats = _tc_argmax(f, cbn_t)
    f_hat = _sc_gather(cbn, idx3.reshape(1, -1))[:, :_WIDTH]
    vq_loss = stats[0, 2]
    vocab_usage = stats[0, 3]
    return (f_hat.reshape(b, l, c), vq_loss, jnp.float32(0.0), vocab_usage)


# speedup vs baseline: 1.1485x; 1.0406x over previous
import functools

import jax
import jax.numpy as jnp
from jax.experimental import pallas as pl
from jax.experimental.pallas import tpu as pltpu
from jax.experimental.pallas import tpu_sc as plsc

_VOCAB = 8192
_WIDTH = 32
_BETA = 0.25
_TOK_BLK = 256
_GATHER_WIN = 128
_GATHER_PAD = 128


def _prep_body(cb_ref, cbn_ref, cbt_ref):
    cb = cb_ref[...]
    ss = jnp.sum(cb * cb, axis=1, keepdims=True)
    r = 1.0 / jnp.maximum(jnp.sqrt(ss), 1e-12)
    cbn = cb * r
    pad = jnp.zeros((_VOCAB, _GATHER_PAD - _WIDTH), jnp.float32)
    cbn_ref[...] = jnp.concatenate([cbn, pad], axis=1)
    cbt_ref[...] = cbn.T


def _argmax_body(nblk, f_a_ref, f_b_ref, cbt_ref, idx_e_ref, idx_o_ref,
                 used_ref, stats_ref, buf_a, buf_b):
    j = pl.program_id(0)
    m_steps = nblk // 2

    @pl.when(j == 0)
    def _():
        stats_ref[0, 0] = 0.0
        stats_ref[0, 1] = 0.0

    def produce(f_ref, buf, fresh):
        f = f_ref[...]
        ss = jnp.sum(f * f, axis=1, keepdims=True)
        fn = f * (1.0 / jnp.maximum(jnp.sqrt(ss), 1e-12))
        buf[...] = jnp.dot(fn, cbt_ref[...],
                           preferred_element_type=jnp.float32)
        stats_ref[0, 1] += jnp.where(fresh, jnp.sum(fn * fn), 0.0)

    def consume(buf, idx_out_ref, fresh):
        s = buf[...]
        m = jnp.max(s, axis=1, keepdims=True)
        mask = s >= m
        iota = jax.lax.broadcasted_iota(jnp.int32, s.shape, 1)
        idx_out_ref[0, 0, :] = jnp.min(jnp.where(mask, iota, _VOCAB), axis=1)
        stats_ref[0, 0] += jnp.where(fresh, jnp.sum(m), 0.0)
        return jnp.max(mask.astype(jnp.float32)
                       .reshape(_TOK_BLK // 8, 8, _VOCAB), axis=0)

    produce(f_a_ref, buf_a, j < m_steps)
    mf_b = consume(buf_b, idx_o_ref, j > 0)
    produce(f_b_ref, buf_b, j < m_steps)
    mf_a = consume(buf_a, idx_e_ref, j < m_steps)

    zero = jnp.zeros((8, _VOCAB), jnp.float32)
    g_b = jnp.where(j > 0, mf_b, zero)
    prev = jnp.where(j == 0, zero, used_ref[...])
    used_ref[...] = jnp.maximum(prev, jnp.maximum(mf_a, g_b))

    @pl.when(j == m_steps)
    def _():
        n = jnp.float32(nblk * _TOK_BLK)
        s_sum = stats_ref[0, 0]
        nf2_sum = stats_ref[0, 1]
        stats_ref[0, 2] = ((1.0 + _BETA) * (nf2_sum - 2.0 * s_sum + n)
                           / (n * _WIDTH))
        used = jnp.max(used_ref[...], axis=0)
        stats_ref[0, 3] = 100.0 * jnp.sum(used) / _VOCAB


def _tc_argmax(f, cbn_t):
    n = f.shape[0]
    nblk = n // _TOK_BLK
    m_steps = nblk // 2
    body = functools.partial(_argmax_body, nblk)
    return pl.pallas_call(
        body,
        grid=(m_steps + 1,),
        in_specs=[
            pl.BlockSpec((_TOK_BLK, _WIDTH),
                         lambda j: (jnp.minimum(2 * j, nblk - 2), 0)),
            pl.BlockSpec((_TOK_BLK, _WIDTH),
                         lambda j: (jnp.minimum(2 * j + 1, nblk - 1), 0)),
            pl.BlockSpec((_WIDTH, _VOCAB), lambda j: (0, 0)),
        ],
        out_specs=[
            pl.BlockSpec((1, 1, _TOK_BLK),
                         lambda j: (jnp.minimum(j, m_steps - 1), 0, 0)),
            pl.BlockSpec((1, 1, _TOK_BLK),
                         lambda j: (jnp.maximum(j - 1, 0), 0, 0)),
            pl.BlockSpec((8, _VOCAB), lambda j: (0, 0)),
            pl.BlockSpec(memory_space=pltpu.SMEM),
        ],
        out_shape=[
            jax.ShapeDtypeStruct((m_steps, 1, _TOK_BLK), jnp.int32),
            jax.ShapeDtypeStruct((m_steps, 1, _TOK_BLK), jnp.int32),
            jax.ShapeDtypeStruct((8, _VOCAB), jnp.float32),
            jax.ShapeDtypeStruct((1, 4), jnp.float32),
        ],
        scratch_shapes=[
            pltpu.VMEM((_TOK_BLK, _VOCAB), jnp.float32),
            pltpu.VMEM((_TOK_BLK, _VOCAB), jnp.float32),
        ],
    )(f, f, cbn_t)


def _prep(cb):
    return pl.pallas_call(
        _prep_body,
        out_shape=[
            jax.ShapeDtypeStruct((_VOCAB, _GATHER_PAD), jnp.float32),
            jax.ShapeDtypeStruct((_WIDTH, _VOCAB), jnp.float32),
        ],
    )(cb)


def _sc_gather(cbn, idx2d):
    n = idx2d.shape[1]
    mesh = plsc.VectorSubcoreMesh(core_axis_name="c", subcore_axis_name="s")

    @functools.partial(
        pl.kernel,
        out_type=jax.ShapeDtypeStruct((n, _GATHER_PAD), jnp.float32),
        mesh=mesh,
    )
    def gather_kernel(cb_hbm, i_hbm, o_hbm):
        def body(i_vmem, o_vmem):
            pltpu.sync_copy(cb_hbm.at[i_vmem.at[0]], o_vmem)

        pltpu.emit_pipeline(
            body,
            grid=(n // _GATHER_WIN,),
            in_specs=[pl.BlockSpec((1, _GATHER_WIN), lambda i: (0, i))],
            out_specs=[pl.BlockSpec((_GATHER_WIN, _GATHER_PAD),
                                    lambda i: (i, 0))],
            core_axis_name="s",
            dimension_semantics=(pltpu.PARALLEL,),
        )(i_hbm, o_hbm)

    return gather_kernel(cbn, idx2d)


def kernel(features, codebook_weight):
    b, l, c = features.shape
    f = features.reshape(-1, c)
    cbn, cbn_t = _prep(codebook_weight)
    idx_e, idx_o, _used, stats = _tc_argmax(f, cbn_t)
    idx = jnp.stack([idx_e[:, 0, :], idx_o[:, 0, :]], axis=1)
    f_hat = _sc_gather(cbn, idx.reshape(1, -1))[:, :_WIDTH]
    vq_loss = stats[0, 2]
    vocab_usage = stats[0, 3]
    return (f_hat.reshape(b, l, c), vq_loss, jnp.float32(0.0), vocab_usage)
